# baseline (speedup 1.0000x reference)
import functools

import jax
import jax.numpy as jnp
from jax import lax
from jax.experimental import pallas as pl
from jax.experimental.pallas import tpu as pltpu
from jax.experimental.pallas import tpu_sc as plsc

_N = 10000
_E = 320000
_D = 128
_NC = 2
_NS = 16
_NW = _NC * _NS
_EPW = _E // _NW
_C = 80
_IPW = _EPW // _C
_NPAD = 10240
_RPT = _NPAD // _NS

_NBLK = 1000
_EBLK = 1600


def _dot(x, w):
    return lax.dot_general(x, w, (((1,), (0,)), ((), ())),
                           precision=lax.Precision.HIGHEST,
                           preferred_element_type=jnp.float32)


def _ln(x, g, beta):
    mu = jnp.mean(x, axis=-1, keepdims=True)
    var = jnp.mean((x - mu) ** 2, axis=-1, keepdims=True)
    return (x - mu) * lax.rsqrt(var + 1e-5) * g + beta


def _hidden(h, w, b, g, beta):
    return jax.nn.relu(_ln(_dot(h, w) + b, g, beta))


def _full(shape):
    return pl.BlockSpec(shape, lambda i: tuple(0 for _ in shape))


def _rows(blk, d):
    return pl.BlockSpec((blk, d), lambda i: (i, 0))


def _prep_mlp(p):
    out = []
    for lyr in p:
        out.append(lyr['W'])
        out.append(lyr['b'][None, :])
        if 'g' in lyr:
            out.append(lyr['g'][None, :])
            out.append(lyr['beta'][None, :])
    return out


def _mlp_ln_body(x, w1, b1, g1, be1, w2, b2, g2, be2, w3, b3, o):
    h = _hidden(x[...], w1[...], b1[...], g1[...], be1[...])
    h = _hidden(h, w2[...], b2[...], g2[...], be2[...])
    o[...] = _dot(h, w3[...]) + b3[...]


def _enc_mlp(x, p, blk):
    n, din = x.shape
    ws = _prep_mlp(p)
    return pl.pallas_call(
        _mlp_ln_body,
        grid=(n // blk,),
        in_specs=[_rows(blk, din)] + [_full(w.shape) for w in ws],
        out_specs=_rows(blk, _D),
        out_shape=jax.ShapeDtypeStruct((n, _D), jnp.float32),
    )(x, *ws)


def _edge_body(want_e, sg, rg, e, ws, wr, we, b1, g1, be1,
               w2, b2, g2, be2, w3, b3, ue_ref, *maybe_en):
    h = _dot(sg[...], ws[...]) + _dot(rg[...], wr[...]) + _dot(e[...], we[...]) + b1[...]
    h = jax.nn.relu(_ln(h, g1[...], be1[...]))
    h = _hidden(h, w2[...], b2[...], g2[...], be2[...])
    ue = _dot(h, w3[...]) + b3[...]
    ue_ref[...] = ue
    if want_e:
        maybe_en[0][...] = e[...] + ue


def _edge_mlp(sg, rg, e, p, want_e):
    w1 = p[0]['W']
    ws_ = [w1[:_D], w1[_D:2 * _D], w1[2 * _D:],
           p[0]['b'][None, :], p[0]['g'][None, :], p[0]['beta'][None, :],
           p[1]['W'], p[1]['b'][None, :], p[1]['g'][None, :], p[1]['beta'][None, :],
           p[2]['W'], p[2]['b'][None, :]]
    out_shape = [jax.ShapeDtypeStruct((_E, _D), jnp.float32)]
    out_specs = [_rows(_EBLK, _D)]
    if want_e:
        out_shape.append(jax.ShapeDtypeStruct((_E, _D), jnp.float32))
        out_specs.append(_rows(_EBLK, _D))
    res = pl.pallas_call(
        functools.partial(_edge_body, want_e),
        grid=(_E // _EBLK,),
        in_specs=[_rows(_EBLK, _D)] * 3 + [_full(w.shape) for w in ws_],
        out_specs=out_specs,
        out_shape=out_shape,
    )(sg, rg, e, *ws_)
    return res if want_e else res[0]


def _node_body(n, a0, a1, wn, wa, b1, g1, be1, w2, b2, g2, be2, w3, b3, o):
    h = _dot(n[...], wn[...]) + _dot(a0[...] + a1[...], wa[...]) + b1[...]
    h = jax.nn.relu(_ln(h, g1[...], be1[...]))
    h = _hidden(h, w2[...], b2[...], g2[...], be2[...])
    o[...] = n[...] + _dot(h, w3[...]) + b3[...]


def _node_mlp(n, a0, a1, p):
    w1 = p[0]['W']
    ws_ = [w1[:_D], w1[_D:],
           p[0]['b'][None, :], p[0]['g'][None, :], p[0]['beta'][None, :],
           p[1]['W'], p[1]['b'][None, :], p[1]['g'][None, :], p[1]['beta'][None, :],
           p[2]['W'], p[2]['b'][None, :]]
    return pl.pallas_call(
        _node_body,
        grid=(_N // _NBLK,),
        in_specs=[_rows(_NBLK, _D)] * 3 + [_full(w.shape) for w in ws_],
        out_specs=_rows(_NBLK, _D),
        out_shape=jax.ShapeDtypeStruct((_N, _D), jnp.float32),
    )(n, a0, a1, *ws_)


def _dec_body(x, w1, b1, w2, b2, w3, b3, o):
    h = jax.nn.relu(_dot(x[...], w1[...]) + b1[...])
    h = jax.nn.relu(_dot(h, w2[...]) + b2[...])
    o[...] = _dot(h, w3[...]) + b3[...]


def _dec_mlp(x, p):
    w3 = jnp.pad(p[2]['W'], ((0, 0), (0, _D - p[2]['W'].shape[1])))
    b3 = jnp.pad(p[2]['b'], (0, _D - p[2]['b'].shape[0]))[None, :]
    ws_ = [p[0]['W'], p[0]['b'][None, :], p[1]['W'], p[1]['b'][None, :], w3, b3]
    out = pl.pallas_call(
        _dec_body,
        grid=(_N // _NBLK,),
        in_specs=[_rows(_NBLK, _D)] + [_full(w.shape) for w in ws_],
        out_specs=_rows(_NBLK, _D),
        out_shape=jax.ShapeDtypeStruct((_N, _D), jnp.float32),
    )(x, *ws_)
    return out[:, :p[2]['W'].shape[1]]


@functools.cache
def _gather_fn():
    mesh = plsc.VectorSubcoreMesh(core_axis_name="c", subcore_axis_name="s",
                                  num_cores=_NC, num_subcores=_NS)

    @functools.partial(
        pl.kernel,
        out_type=(jax.ShapeDtypeStruct((_E, _D), jnp.float32),
                  jax.ShapeDtypeStruct((_E, _D), jnp.float32)),
        mesh=mesh,
        scratch_types=[
            pltpu.VMEM((_IPW, _C), jnp.int32),
            pltpu.VMEM((_IPW, _C), jnp.int32),
            pltpu.VMEM((_C, _D), jnp.float32),
            pltpu.VMEM((_C, _D), jnp.float32),
            pltpu.SemaphoreType.DMA,
            pltpu.SemaphoreType.DMA,
        ],
    )
    def _sc_gather(n_hbm, s_hbm, r_hbm, sg_hbm, rg_hbm,
                   sidx, ridx, sbuf, rbuf, sem1, sem2):
        wid = lax.axis_index("s") * _NC + lax.axis_index("c")
        ebase = wid * _EPW
        pltpu.sync_copy(s_hbm.at[wid], sidx)
        pltpu.sync_copy(r_hbm.at[wid], ridx)

        def body(j, carry):
            off = ebase + j * _C
            cp1 = pltpu.async_copy(n_hbm.at[sidx.at[j]], sbuf, sem1)
            cp2 = pltpu.async_copy(n_hbm.at[ridx.at[j]], rbuf, sem2)
            cp1.wait()
            cp2.wait()
            pltpu.sync_copy(sbuf, sg_hbm.at[pl.ds(off, _C)])
            pltpu.sync_copy(rbuf, rg_hbm.at[pl.ds(off, _C)])
            return carry

        lax.fori_loop(0, _IPW, body, 0)

    return _sc_gather


@functools.cache
def _scatter_fn():
    mesh = plsc.VectorSubcoreMesh(core_axis_name="c", subcore_axis_name="s",
                                  num_cores=_NC, num_subcores=_NS)

    @functools.partial(
        pl.kernel,
        out_type=jax.ShapeDtypeStruct((_NC, _NPAD, _D), jnp.float32),
        mesh=mesh,
        scratch_types=[
            pltpu.VMEM((_IPW, _C), jnp.int32),
            pltpu.VMEM((_C, _D), jnp.float32),
            pltpu.VMEM_SHARED((_NPAD, _D), jnp.float32),
            pltpu.SemaphoreType.DMA,
        ],
    )
    def _sc_scatter(ue_hbm, r_hbm, out_hbm, ridx, ebuf, acc, sem):
        cid = lax.axis_index("c")
        sid = lax.axis_index("s")
        wid = sid * _NC + cid

        def zrow(i, carry):
            for kk in range(_D // 16):
                ebuf[i, pl.ds(kk * 16, 16)] = jnp.zeros((16,), jnp.float32)
            return carry

        lax.fori_loop(0, _C, zrow, 0)
        for z in range(_RPT // _C):
            pltpu.sync_copy(ebuf, acc.at[pl.ds(sid * _RPT + z * _C, _C)])
        plsc.subcore_barrier()

        ebase = wid * _EPW
        pltpu.sync_copy(r_hbm.at[wid], ridx)

        def body(j, carry):
            pltpu.sync_copy(ue_hbm.at[pl.ds(ebase + j * _C, _C)], ebuf)
            pltpu.sync_copy(ebuf, acc.at[ridx.at[j]], add=True)
            return carry

        lax.fori_loop(0, _IPW, body, 0)
        plsc.subcore_barrier()
        pltpu.sync_copy(acc.at[pl.ds(sid * _RPT, _RPT)],
                        out_hbm.at[cid, pl.ds(sid * _RPT, _RPT)])

    return _sc_scatter


def kernel(nodes, edges, senders, receivers, params):
    send2d = senders.astype(jnp.int32).reshape(_NW, _IPW, _C)
    recv2d = receivers.astype(jnp.int32).reshape(_NW, _IPW, _C)

    n = _enc_mlp(nodes, params['enc_node'], _NBLK)
    e = _enc_mlp(edges, params['enc_edge'], _EBLK)

    for si, step in enumerate(params['proc']):
        sg, rg = _gather_fn()(n, send2d, recv2d)
        if si == 0:
            ue, e = _edge_mlp(sg, rg, e, step['edge'], want_e=True)
        else:
            ue = _edge_mlp(sg, rg, e, step['edge'], want_e=False)
        agg = _scatter_fn()(ue, recv2d)
        n = _node_mlp(n, agg[0, :_N], agg[1, :_N], step['node'])

    return _dec_mlp(n, params['dec'])


# speedup vs baseline: 2.4690x; 2.4690x over previous
import functools

import jax
import jax.numpy as jnp
from jax import lax
from jax.experimental import pallas as pl
from jax.experimental.pallas import tpu as pltpu
from jax.experimental.pallas import tpu_sc as plsc

_N = 10000
_E = 320000
_D = 128
_NC = 2
_NS = 16
_NW = _NC * _NS
_C = 80
_NPAD = 10240
_RPT = _NPAD // _NS

_NBLK = 1000
_EBLK = 1600

_CHUNKS = ((0, 60), (153600, 65))


def _csize(ipw):
    return _NW * ipw * _C


def _dot(x, w):
    return lax.dot_general(x, w, (((1,), (0,)), ((), ())),
                           preferred_element_type=jnp.float32)


def _ln(x, g, beta):
    mu = jnp.mean(x, axis=-1, keepdims=True)
    var = jnp.mean((x - mu) ** 2, axis=-1, keepdims=True)
    return (x - mu) * lax.rsqrt(var + 1e-5) * g + beta


def _hidden(h, w, b, g, beta):
    return jax.nn.relu(_ln(_dot(h, w) + b, g, beta))


def _full(shape):
    return pl.BlockSpec(shape, lambda i: tuple(0 for _ in shape))


def _rows(blk, d, off=0):
    return pl.BlockSpec((blk, d), lambda i: (off + i, 0))


def _prep_mlp(p):
    out = []
    for lyr in p:
        out.append(lyr['W'])
        out.append(lyr['b'][None, :])
        if 'g' in lyr:
            out.append(lyr['g'][None, :])
            out.append(lyr['beta'][None, :])
    return out


def _mlp_ln_body(x, w1, b1, g1, be1, w2, b2, g2, be2, w3, b3, o):
    h = _hidden(x[...], w1[...], b1[...], g1[...], be1[...])
    h = _hidden(h, w2[...], b2[...], g2[...], be2[...])
    o[...] = _dot(h, w3[...]) + b3[...]


def _enc_mlp(x, p, blk):
    n, din = x.shape
    ws = _prep_mlp(p)
    return pl.pallas_call(
        _mlp_ln_body,
        grid=(n // blk,),
        in_specs=[_rows(blk, din)] + [_full(w.shape) for w in ws],
        out_specs=_rows(blk, _D),
        out_shape=jax.ShapeDtypeStruct((n, _D), jnp.float32),
    )(x, *ws)


def _edge_body(want_e, sg, rg, e, ws, wr, we, b1, g1, be1,
               w2, b2, g2, be2, w3, b3, ue_ref, *maybe_en):
    h = _dot(sg[...], ws[...]) + _dot(rg[...], wr[...]) + _dot(e[...], we[...]) + b1[...]
    h = jax.nn.relu(_ln(h, g1[...], be1[...]))
    h = _hidden(h, w2[...], b2[...], g2[...], be2[...])
    ue = _dot(h, w3[...]) + b3[...]
    ue_ref[...] = ue
    if want_e:
        maybe_en[0][...] = e[...] + ue


def _edge_mlp(sg, rg, e, e_off, p, want_e):
    ec = sg.shape[0]
    w1 = p[0]['W']
    ws_ = [w1[:_D], w1[_D:2 * _D], w1[2 * _D:],
           p[0]['b'][None, :], p[0]['g'][None, :], p[0]['beta'][None, :],
           p[1]['W'], p[1]['b'][None, :], p[1]['g'][None, :], p[1]['beta'][None, :],
           p[2]['W'], p[2]['b'][None, :]]
    out_shape = [jax.ShapeDtypeStruct((ec, _D), jnp.float32)]
    out_specs = [_rows(_EBLK, _D)]
    if want_e:
        out_shape.append(jax.ShapeDtypeStruct((ec, _D), jnp.float32))
        out_specs.append(_rows(_EBLK, _D))
    res = pl.pallas_call(
        functools.partial(_edge_body, want_e),
        grid=(ec // _EBLK,),
        in_specs=[_rows(_EBLK, _D)] * 2 + [_rows(_EBLK, _D, e_off)]
                 + [_full(w.shape) for w in ws_],
        out_specs=out_specs,
        out_shape=out_shape,
    )(sg, rg, e, *ws_)
    return res if want_e else res[0]


def _node_body(n, a0, a1, wn, wa, b1, g1, be1, w2, b2, g2, be2, w3, b3, o):
    agg = a0[0] + a0[1] + a1[0] + a1[1]
    h = _dot(n[...], wn[...]) + _dot(agg, wa[...]) + b1[...]
    h = jax.nn.relu(_ln(h, g1[...], be1[...]))
    h = _hidden(h, w2[...], b2[...], g2[...], be2[...])
    o[...] = n[...] + _dot(h, w3[...]) + b3[...]


def _node_mlp(n, aggs, p):
    w1 = p[0]['W']
    ws_ = [w1[:_D], w1[_D:],
           p[0]['b'][None, :], p[0]['g'][None, :], p[0]['beta'][None, :],
           p[1]['W'], p[1]['b'][None, :], p[1]['g'][None, :], p[1]['beta'][None, :],
           p[2]['W'], p[2]['b'][None, :]]
    agg_spec = pl.BlockSpec((_NC, _NBLK, _D), lambda i: (0, i, 0))
    return pl.pallas_call(
        _node_body,
        grid=(_N // _NBLK,),
        in_specs=[_rows(_NBLK, _D)] + [agg_spec] * len(aggs)
                 + [_full(w.shape) for w in ws_],
        out_specs=_rows(_NBLK, _D),
        out_shape=jax.ShapeDtypeStruct((_N, _D), jnp.float32),
    )(n, *aggs, *ws_)


def _dec_body(x, w1, b1, w2, b2, w3, b3, o):
    h = jax.nn.relu(_dot(x[...], w1[...]) + b1[...])
    h = jax.nn.relu(_dot(h, w2[...]) + b2[...])
    o[...] = _dot(h, w3[...]) + b3[...]


def _dec_mlp(x, p):
    w3 = jnp.pad(p[2]['W'], ((0, 0), (0, _D - p[2]['W'].shape[1])))
    b3 = jnp.pad(p[2]['b'], (0, _D - p[2]['b'].shape[0]))[None, :]
    ws_ = [p[0]['W'], p[0]['b'][None, :], p[1]['W'], p[1]['b'][None, :], w3, b3]
    out = pl.pallas_call(
        _dec_body,
        grid=(_N // _NBLK,),
        in_specs=[_rows(_NBLK, _D)] + [_full(w.shape) for w in ws_],
        out_specs=_rows(_NBLK, _D),
        out_shape=jax.ShapeDtypeStruct((_N, _D), jnp.float32),
    )(x, *ws_)
    return out[:, :p[2]['W'].shape[1]]


def _sc_mesh():
    return plsc.VectorSubcoreMesh(core_axis_name="c", subcore_axis_name="s",
                                  num_cores=_NC, num_subcores=_NS)


@functools.cache
def _gather_fn(ipw):
    ec = _csize(ipw)

    @functools.partial(
        pl.kernel,
        out_type=(jax.ShapeDtypeStruct((ec, _D), jnp.float32),
                  jax.ShapeDtypeStruct((ec, _D), jnp.float32)),
        mesh=_sc_mesh(),
        scratch_types=[
            pltpu.VMEM((ipw, _C), jnp.int32),
            pltpu.VMEM((ipw, _C), jnp.int32),
            pltpu.VMEM((_C, _D), jnp.float32),
            pltpu.VMEM((_C, _D), jnp.float32),
            pltpu.SemaphoreType.DMA,
            pltpu.SemaphoreType.DMA,
        ],
    )
    def _sc_gather(n_hbm, s_hbm, r_hbm, sg_hbm, rg_hbm,
                   sidx, ridx, sbuf, rbuf, sem1, sem2):
        wid = lax.axis_index("s") * _NC + lax.axis_index("c")
        ebase = wid * ipw * _C
        pltpu.sync_copy(s_hbm.at[wid], sidx)
        pltpu.sync_copy(r_hbm.at[wid], ridx)

        def body(j, carry):
            off = ebase + j * _C
            cp1 = pltpu.async_copy(n_hbm.at[sidx.at[j]], sbuf, sem1)
            cp2 = pltpu.async_copy(n_hbm.at[ridx.at[j]], rbuf, sem2)
            cp1.wait()
            cp2.wait()
            pltpu.sync_copy(sbuf, sg_hbm.at[pl.ds(off, _C)])
            pltpu.sync_copy(rbuf, rg_hbm.at[pl.ds(off, _C)])
            return carry

        lax.fori_loop(0, ipw, body, 0)

    return _sc_gather


@functools.cache
def _scatter_fn(ipw):
    ec = _csize(ipw)

    @functools.partial(
        pl.kernel,
        out_type=jax.ShapeDtypeStruct((_NC, _NPAD, _D), jnp.float32),
        mesh=_sc_mesh(),
        scratch_types=[
            pltpu.VMEM((ipw, _C), jnp.int32),
            pltpu.VMEM((_C, _D), jnp.float32),
            pltpu.VMEM_SHARED((_NPAD, _D), jnp.float32),
            pltpu.SemaphoreType.DMA,
        ],
    )
    def _sc_scatter(ue_hbm, r_hbm, out_hbm, ridx, ebuf, acc, sem):
        cid = lax.axis_index("c")
        sid = lax.axis_index("s")
        wid = sid * _NC + cid

        def zrow(i, carry):
            for kk in range(_D // 16):
                ebuf[i, pl.ds(kk * 16, 16)] = jnp.zeros((16,), jnp.float32)
            return carry

        lax.fori_loop(0, _C, zrow, 0)
        for z in range(_RPT // _C):
            pltpu.sync_copy(ebuf, acc.at[pl.ds(sid * _RPT + z * _C, _C)])
        plsc.subcore_barrier()

        ebase = wid * ipw * _C
        pltpu.sync_copy(r_hbm.at[wid], ridx)

        def body(j, carry):
            pltpu.sync_copy(ue_hbm.at[pl.ds(ebase + j * _C, _C)], ebuf)
            pltpu.sync_copy(ebuf, acc.at[ridx.at[j]], add=True)
            return carry

        lax.fori_loop(0, ipw, body, 0)
        plsc.subcore_barrier()
        pltpu.sync_copy(acc.at[pl.ds(sid * _RPT, _RPT)],
                        out_hbm.at[cid, pl.ds(sid * _RPT, _RPT)])

    return _sc_scatter


def kernel(nodes, edges, senders, receivers, params):
    senders = senders.astype(jnp.int32)
    receivers = receivers.astype(jnp.int32)
    sidx = [senders[b:b + _csize(ipw)].reshape(_NW, ipw, _C) for b, ipw in _CHUNKS]
    ridx = [receivers[b:b + _csize(ipw)].reshape(_NW, ipw, _C) for b, ipw in _CHUNKS]

    n = _enc_mlp(nodes, params['enc_node'], _NBLK)
    e = _enc_mlp(edges, params['enc_edge'], _EBLK)

    nsteps = len(params['proc'])
    e_chunks = None
    for si, step in enumerate(params['proc']):
        want_e = si + 1 < nsteps
        aggs = []
        new_e_chunks = []
        for q, (b, ipw) in enumerate(_CHUNKS):
            sg, rg = _gather_fn(ipw)(n, sidx[q], ridx[q])
            if e_chunks is None:
                eq_src, eq_off = e, b // _EBLK
            else:
                eq_src, eq_off = e_chunks[q], 0
            res = _edge_mlp(sg, rg, eq_src, eq_off, step['edge'], want_e)
            if want_e:
                ue, en = res
                new_e_chunks.append(en)
            else:
                ue = res
            aggs.append(_scatter_fn(ipw)(ue, ridx[q]))
        e_chunks = new_e_chunks if want_e else e_chunks
        n = _node_mlp(n, aggs, step['node'])

    return _dec_mlp(n, params['dec'])


# speedup vs baseline: 2.5461x; 1.0312x over previous
import functools

import jax
import jax.numpy as jnp
from jax import lax
from jax.experimental import pallas as pl
from jax.experimental.pallas import tpu as pltpu
from jax.experimental.pallas import tpu_sc as plsc

_N = 10000
_E = 320000
_D = 128
_NC = 2
_NS = 16
_NW = _NC * _NS
_C = 80
_NPAD = 10240
_RPT = _NPAD // _NS

_NBLK = 1000
_EBLK = 1600

_CHUNKS = ((0, 60), (153600, 65))


def _csize(ipw):
    return _NW * ipw * _C


def _dot(x, w):
    return lax.dot_general(x, w, (((1,), (0,)), ((), ())),
                           preferred_element_type=jnp.float32)


def _ln(x, g, beta):
    mu = jnp.mean(x, axis=-1, keepdims=True)
    var = jnp.mean((x - mu) ** 2, axis=-1, keepdims=True)
    return (x - mu) * lax.rsqrt(var + 1e-5) * g + beta


def _hidden(h, w, b, g, beta):
    return jax.nn.relu(_ln(_dot(h, w) + b, g, beta))


def _full(shape):
    return pl.BlockSpec(shape, lambda i: tuple(0 for _ in shape))


def _rows(blk, d, off=0):
    return pl.BlockSpec((blk, d), lambda i: (off + i, 0))


def _prep_mlp(p):
    out = []
    for lyr in p:
        out.append(lyr['W'])
        out.append(lyr['b'][None, :])
        if 'g' in lyr:
            out.append(lyr['g'][None, :])
            out.append(lyr['beta'][None, :])
    return out


def _mlp_ln_body(x, w1, b1, g1, be1, w2, b2, g2, be2, w3, b3, o):
    h = _hidden(x[...], w1[...], b1[...], g1[...], be1[...])
    h = _hidden(h, w2[...], b2[...], g2[...], be2[...])
    o[...] = _dot(h, w3[...]) + b3[...]


def _enc_mlp(x, p, blk):
    n, din = x.shape
    ws = _prep_mlp(p)
    return pl.pallas_call(
        _mlp_ln_body,
        grid=(n // blk,),
        in_specs=[_rows(blk, din)] + [_full(w.shape) for w in ws],
        out_specs=_rows(blk, _D),
        out_shape=jax.ShapeDtypeStruct((n, _D), jnp.float32),
    )(x, *ws)


def _edge_body(want_e, sg, rg, e, ws, wr, we, b1, g1, be1,
               w2, b2, g2, be2, w3, b3, ue_ref, *maybe_en):
    h = _dot(sg[...], ws[...]) + _dot(rg[...], wr[...]) + _dot(e[...], we[...]) + b1[...]
    h = jax.nn.relu(_ln(h, g1[...], be1[...]))
    h = _hidden(h, w2[...], b2[...], g2[...], be2[...])
    ue = _dot(h, w3[...]) + b3[...]
    ue_ref[...] = ue
    if want_e:
        maybe_en[0][...] = e[...] + ue


def _edge_mlp(sg, rg, e, e_off, p, want_e):
    ec = sg.shape[0]
    w1 = p[0]['W']
    ws_ = [w1[:_D], w1[_D:2 * _D], w1[2 * _D:],
           p[0]['b'][None, :], p[0]['g'][None, :], p[0]['beta'][None, :],
           p[1]['W'], p[1]['b'][None, :], p[1]['g'][None, :], p[1]['beta'][None, :],
           p[2]['W'], p[2]['b'][None, :]]
    out_shape = [jax.ShapeDtypeStruct((ec, _D), jnp.float32)]
    out_specs = [_rows(_EBLK, _D)]
    if want_e:
        out_shape.append(jax.ShapeDtypeStruct((ec, _D), jnp.float32))
        out_specs.append(_rows(_EBLK, _D))
    res = pl.pallas_call(
        functools.partial(_edge_body, want_e),
        grid=(ec // _EBLK,),
        in_specs=[_rows(_EBLK, _D)] * 2 + [_rows(_EBLK, _D, e_off)]
                 + [_full(w.shape) for w in ws_],
        out_specs=out_specs,
        out_shape=out_shape,
    )(sg, rg, e, *ws_)
    return res if want_e else res[0]


def _node_body(n, a0, a1, wn, wa, b1, g1, be1, w2, b2, g2, be2, w3, b3, o):
    agg = a0[0] + a0[1] + a1[0] + a1[1]
    h = _dot(n[...], wn[...]) + _dot(agg, wa[...]) + b1[...]
    h = jax.nn.relu(_ln(h, g1[...], be1[...]))
    h = _hidden(h, w2[...], b2[...], g2[...], be2[...])
    o[...] = n[...] + _dot(h, w3[...]) + b3[...]


def _node_mlp(n, aggs, p):
    w1 = p[0]['W']
    ws_ = [w1[:_D], w1[_D:],
           p[0]['b'][None, :], p[0]['g'][None, :], p[0]['beta'][None, :],
           p[1]['W'], p[1]['b'][None, :], p[1]['g'][None, :], p[1]['beta'][None, :],
           p[2]['W'], p[2]['b'][None, :]]
    agg_spec = pl.BlockSpec((_NC, _NBLK, _D), lambda i: (0, i, 0))
    return pl.pallas_call(
        _node_body,
        grid=(_N // _NBLK,),
        in_specs=[_rows(_NBLK, _D)] + [agg_spec] * len(aggs)
                 + [_full(w.shape) for w in ws_],
        out_specs=_rows(_NBLK, _D),
        out_shape=jax.ShapeDtypeStruct((_N, _D), jnp.float32),
    )(n, *aggs, *ws_)


def _dec_body(x, w1, b1, w2, b2, w3, b3, o):
    h = jax.nn.relu(_dot(x[...], w1[...]) + b1[...])
    h = jax.nn.relu(_dot(h, w2[...]) + b2[...])
    o[...] = _dot(h, w3[...]) + b3[...]


def _dec_mlp(x, p):
    w3 = jnp.pad(p[2]['W'], ((0, 0), (0, _D - p[2]['W'].shape[1])))
    b3 = jnp.pad(p[2]['b'], (0, _D - p[2]['b'].shape[0]))[None, :]
    ws_ = [p[0]['W'], p[0]['b'][None, :], p[1]['W'], p[1]['b'][None, :], w3, b3]
    out = pl.pallas_call(
        _dec_body,
        grid=(_N // _NBLK,),
        in_specs=[_rows(_NBLK, _D)] + [_full(w.shape) for w in ws_],
        out_specs=_rows(_NBLK, _D),
        out_shape=jax.ShapeDtypeStruct((_N, _D), jnp.float32),
    )(x, *ws_)
    return out[:, :p[2]['W'].shape[1]]


def _sc_mesh():
    return plsc.VectorSubcoreMesh(core_axis_name="c", subcore_axis_name="s",
                                  num_cores=_NC, num_subcores=_NS)


@functools.cache
def _gather_fn(ipw):
    ec = _csize(ipw)

    @functools.partial(
        pl.kernel,
        out_type=(jax.ShapeDtypeStruct((ec, _D), jnp.float32),
                  jax.ShapeDtypeStruct((ec, _D), jnp.float32)),
        mesh=_sc_mesh(),
        scratch_types=[
            pltpu.VMEM((ipw, _C), jnp.int32),
            pltpu.VMEM((ipw, _C), jnp.int32),
            pltpu.VMEM((2, 2, _C, _D), jnp.float32),
            pltpu.SemaphoreType.DMA,
            pltpu.SemaphoreType.DMA,
        ],
    )
    def _sc_gather(n_hbm, s_hbm, r_hbm, sg_hbm, rg_hbm,
                   sidx, ridx, bufs, gsem, wsem):
        wid = lax.axis_index("s") * _NC + lax.axis_index("c")
        ebase = wid * ipw * _C
        pltpu.sync_copy(s_hbm.at[wid], sidx)
        pltpu.sync_copy(r_hbm.at[wid], ridx)

        def gath(j, b):
            pltpu.async_copy(n_hbm.at[sidx.at[j]], bufs.at[b, 0], gsem)
            pltpu.async_copy(n_hbm.at[ridx.at[j]], bufs.at[b, 1], gsem)

        def gath_wait(j, b):
            pltpu.make_async_copy(n_hbm.at[sidx.at[j]], bufs.at[b, 0], gsem).wait()
            pltpu.make_async_copy(n_hbm.at[ridx.at[j]], bufs.at[b, 1], gsem).wait()

        def wr(j, b):
            off = ebase + j * _C
            pltpu.async_copy(bufs.at[b, 0], sg_hbm.at[pl.ds(off, _C)], wsem)
            pltpu.async_copy(bufs.at[b, 1], rg_hbm.at[pl.ds(off, _C)], wsem)

        def wr_wait(j, b):
            off = ebase + j * _C
            pltpu.make_async_copy(bufs.at[b, 0], sg_hbm.at[pl.ds(off, _C)], wsem).wait()
            pltpu.make_async_copy(bufs.at[b, 1], rg_hbm.at[pl.ds(off, _C)], wsem).wait()

        gath(0, 0)

        def body(j2, carry):
            a = 2 * j2
            gath_wait(a, 0)

            @pl.when(j2 > 0)
            def _():
                wr_wait(a - 1, 1)

            wr(a, 0)
            gath(a + 1, 1)
            gath_wait(a + 1, 1)
            wr_wait(a, 0)
            wr(a + 1, 1)

            @pl.when(a + 2 < ipw)
            def _():
                gath(a + 2, 0)

            return carry

        lax.fori_loop(0, ipw // 2, body, 0)
        if ipw % 2:
            gath_wait(ipw - 1, 0)
            wr_wait(ipw - 2, 1)
            wr(ipw - 1, 0)
            wr_wait(ipw - 1, 0)
        else:
            wr_wait(ipw - 1, 1)

    return _sc_gather


@functools.cache
def _scatter_fn(ipw):
    ec = _csize(ipw)

    @functools.partial(
        pl.kernel,
        out_type=jax.ShapeDtypeStruct((_NC, _NPAD, _D), jnp.float32),
        mesh=_sc_mesh(),
        scratch_types=[
            pltpu.VMEM((ipw, _C), jnp.int32),
            pltpu.VMEM((2, _C, _D), jnp.float32),
            pltpu.VMEM_SHARED((_NPAD, _D), jnp.float32),
            pltpu.SemaphoreType.DMA,
        ],
    )
    def _sc_scatter(ue_hbm, r_hbm, out_hbm, ridx, ebuf, acc, rsem):
        cid = lax.axis_index("c")
        sid = lax.axis_index("s")
        wid = sid * _NC + cid

        def zrow(i, carry):
            for kk in range(_D // 16):
                ebuf[0, i, pl.ds(kk * 16, 16)] = jnp.zeros((16,), jnp.float32)
            return carry

        lax.fori_loop(0, _C, zrow, 0)
        for z in range(_RPT // _C):
            pltpu.sync_copy(ebuf.at[0], acc.at[pl.ds(sid * _RPT + z * _C, _C)])
        plsc.subcore_barrier()

        ebase = wid * ipw * _C
        pltpu.sync_copy(r_hbm.at[wid], ridx)

        def rd(j, b):
            pltpu.async_copy(ue_hbm.at[pl.ds(ebase + j * _C, _C)], ebuf.at[b], rsem)

        def rd_wait(j, b):
            pltpu.make_async_copy(ue_hbm.at[pl.ds(ebase + j * _C, _C)],
                                  ebuf.at[b], rsem).wait()

        rd(0, 0)

        def body(j2, carry):
            a = 2 * j2
            rd_wait(a, 0)
            rd(a + 1, 1)
            pltpu.sync_copy(ebuf.at[0], acc.at[ridx.at[a]], add=True)
            rd_wait(a + 1, 1)

            @pl.when(a + 2 < ipw)
            def _():
                rd(a + 2, 0)

            pltpu.sync_copy(ebuf.at[1], acc.at[ridx.at[a + 1]], add=True)
            return carry

        lax.fori_loop(0, ipw // 2, body, 0)
        if ipw % 2:
            rd_wait(ipw - 1, 0)
            pltpu.sync_copy(ebuf.at[0], acc.at[ridx.at[ipw - 1]], add=True)
        plsc.subcore_barrier()
        pltpu.sync_copy(acc.at[pl.ds(sid * _RPT, _RPT)],
                        out_hbm.at[cid, pl.ds(sid * _RPT, _RPT)])

    return _sc_scatter


def kernel(nodes, edges, senders, receivers, params):
    senders = senders.astype(jnp.int32)
    receivers = receivers.astype(jnp.int32)
    sidx = [senders[b:b + _csize(ipw)].reshape(_NW, ipw, _C) for b, ipw in _CHUNKS]
    ridx = [receivers[b:b + _csize(ipw)].reshape(_NW, ipw, _C) for b, ipw in _CHUNKS]

    n = _enc_mlp(nodes, params['enc_node'], _NBLK)
    e = _enc_mlp(edges, params['enc_edge'], _EBLK)

    nsteps = len(params['proc'])
    e_chunks = None
    for si, step in enumerate(params['proc']):
        want_e = si + 1 < nsteps
        aggs = []
        new_e_chunks = []
        for q, (b, ipw) in enumerate(_CHUNKS):
            sg, rg = _gather_fn(ipw)(n, sidx[q], ridx[q])
            if e_chunks is None:
                eq_src, eq_off = e, b // _EBLK
            else:
                eq_src, eq_off = e_chunks[q], 0
            res = _edge_mlp(sg, rg, eq_src, eq_off, step['edge'], want_e)
            if want_e:
                ue, en = res
                new_e_chunks.append(en)
            else:
                ue = res
            aggs.append(_scatter_fn(ipw)(ue, ridx[q]))
        e_chunks = new_e_chunks if want_e else e_chunks
        n = _node_mlp(n, aggs, step['node'])

    return _dec_mlp(n, params['dec'])


# speedup vs baseline: 2.8775x; 1.1302x over previous
import functools

import jax
import jax.numpy as jnp
from jax import lax
from jax.experimental import pallas as pl
from jax.experimental.pallas import tpu as pltpu
from jax.experimental.pallas import tpu_sc as plsc

_N = 10000
_E = 320000
_D = 128
_NC = 2
_NS = 16
_NW = _NC * _NS
_C = 80
_NPAD = 10240
_RPT = _NPAD // _NS

_NBLK = 1000
_EBLK = 1600

_CHUNKS = ((0, 60), (153600, 65))


def _csize(ipw):
    return _NW * ipw * _C


def _dot(x, w):
    return lax.dot_general(x, w, (((1,), (0,)), ((), ())),
                           preferred_element_type=jnp.float32)


def _ln(x, g, beta):
    mu = jnp.mean(x, axis=-1, keepdims=True)
    var = jnp.mean((x - mu) ** 2, axis=-1, keepdims=True)
    return (x - mu) * lax.rsqrt(var + 1e-5) * g + beta


def _hidden(h, w, b, g, beta):
    return jax.nn.relu(_ln(_dot(h, w) + b, g, beta))


def _full(shape):
    return pl.BlockSpec(shape, lambda i: tuple(0 for _ in shape))


def _rows(blk, d, off=0):
    return pl.BlockSpec((blk, d), lambda i: (off + i, 0))


def _prep_mlp(p):
    out = []
    for lyr in p:
        out.append(lyr['W'])
        out.append(lyr['b'][None, :])
        if 'g' in lyr:
            out.append(lyr['g'][None, :])
            out.append(lyr['beta'][None, :])
    return out


def _mlp_ln_body(x, w1, b1, g1, be1, w2, b2, g2, be2, w3, b3, o):
    h = _hidden(x[...], w1[...], b1[...], g1[...], be1[...])
    h = _hidden(h, w2[...], b2[...], g2[...], be2[...])
    o[...] = _dot(h, w3[...]) + b3[...]


def _enc_mlp(x, p, blk):
    n, din = x.shape
    ws = _prep_mlp(p)
    return pl.pallas_call(
        _mlp_ln_body,
        grid=(n // blk,),
        in_specs=[_rows(blk, din)] + [_full(w.shape) for w in ws],
        out_specs=_rows(blk, _D),
        out_shape=jax.ShapeDtypeStruct((n, _D), jnp.float32),
    )(x, *ws)


def _edge_body(want_e, n_enc, sg, rg, e, *refs):
    if n_enc:
        encw, refs = refs[:n_enc], refs[n_enc:]
        e0 = e[...]
        e0 = _hidden(e0, encw[0][...], encw[1][...], encw[2][...], encw[3][...])
        e0 = _hidden(e0, encw[4][...], encw[5][...], encw[6][...], encw[7][...])
        e0 = _dot(e0, encw[8][...]) + encw[9][...]
    else:
        e0 = e[...]
    (ws, wr, we, b1, g1, be1, w2, b2, g2, be2, w3, b3), refs = refs[:12], refs[12:]
    h = _dot(sg[...], ws[...]) + _dot(rg[...], wr[...]) + _dot(e0, we[...]) + b1[...]
    h = jax.nn.relu(_ln(h, g1[...], be1[...]))
    h = _hidden(h, w2[...], b2[...], g2[...], be2[...])
    ue = _dot(h, w3[...]) + b3[...]
    refs[0][...] = ue
    if want_e:
        refs[1][...] = e0 + ue


def _edge_mlp(sg, rg, e, e_off, p, want_e, enc_p=None):
    ec = sg.shape[0]
    ed = e.shape[1]
    encw = _prep_mlp(enc_p) if enc_p is not None else []
    w1 = p[0]['W']
    ws_ = encw + [w1[:_D], w1[_D:2 * _D], w1[2 * _D:],
                  p[0]['b'][None, :], p[0]['g'][None, :], p[0]['beta'][None, :],
                  p[1]['W'], p[1]['b'][None, :], p[1]['g'][None, :], p[1]['beta'][None, :],
                  p[2]['W'], p[2]['b'][None, :]]
    out_shape = [jax.ShapeDtypeStruct((ec, _D), jnp.float32)]
    out_specs = [_rows(_EBLK, _D)]
    if want_e:
        out_shape.append(jax.ShapeDtypeStruct((ec, _D), jnp.float32))
        out_specs.append(_rows(_EBLK, _D))
    res = pl.pallas_call(
        functools.partial(_edge_body, want_e, len(encw)),
        grid=(ec // _EBLK,),
        in_specs=[_rows(_EBLK, _D)] * 2 + [_rows(_EBLK, ed, e_off)]
                 + [_full(w.shape) for w in ws_],
        out_specs=out_specs,
        out_shape=out_shape,
    )(sg, rg, e, *ws_)
    return res if want_e else res[0]


def _node_body(n, a0, a1, wn, wa, b1, g1, be1, w2, b2, g2, be2, w3, b3, o):
    agg = a0[0] + a0[1] + a1[0] + a1[1]
    h = _dot(n[...], wn[...]) + _dot(agg, wa[...]) + b1[...]
    h = jax.nn.relu(_ln(h, g1[...], be1[...]))
    h = _hidden(h, w2[...], b2[...], g2[...], be2[...])
    o[...] = n[...] + _dot(h, w3[...]) + b3[...]


def _node_mlp(n, aggs, p):
    w1 = p[0]['W']
    ws_ = [w1[:_D], w1[_D:],
           p[0]['b'][None, :], p[0]['g'][None, :], p[0]['beta'][None, :],
           p[1]['W'], p[1]['b'][None, :], p[1]['g'][None, :], p[1]['beta'][None, :],
           p[2]['W'], p[2]['b'][None, :]]
    agg_spec = pl.BlockSpec((_NC, _NBLK, _D), lambda i: (0, i, 0))
    return pl.pallas_call(
        _node_body,
        grid=(_N // _NBLK,),
        in_specs=[_rows(_NBLK, _D)] + [agg_spec] * len(aggs)
                 + [_full(w.shape) for w in ws_],
        out_specs=_rows(_NBLK, _D),
        out_shape=jax.ShapeDtypeStruct((_N, _D), jnp.float32),
    )(n, *aggs, *ws_)


def _dec_body(x, w1, b1, w2, b2, w3, b3, o):
    h = jax.nn.relu(_dot(x[...], w1[...]) + b1[...])
    h = jax.nn.relu(_dot(h, w2[...]) + b2[...])
    o[...] = _dot(h, w3[...]) + b3[...]


def _dec_mlp(x, p):
    w3 = jnp.pad(p[2]['W'], ((0, 0), (0, _D - p[2]['W'].shape[1])))
    b3 = jnp.pad(p[2]['b'], (0, _D - p[2]['b'].shape[0]))[None, :]
    ws_ = [p[0]['W'], p[0]['b'][None, :], p[1]['W'], p[1]['b'][None, :], w3, b3]
    out = pl.pallas_call(
        _dec_body,
        grid=(_N // _NBLK,),
        in_specs=[_rows(_NBLK, _D)] + [_full(w.shape) for w in ws_],
        out_specs=_rows(_NBLK, _D),
        out_shape=jax.ShapeDtypeStruct((_N, _D), jnp.float32),
    )(x, *ws_)
    return out[:, :p[2]['W'].shape[1]]


def _sc_mesh():
    return plsc.VectorSubcoreMesh(core_axis_name="c", subcore_axis_name="s",
                                  num_cores=_NC, num_subcores=_NS)


@functools.cache
def _gather_fn(ipw):
    ec = _csize(ipw)

    @functools.partial(
        pl.kernel,
        out_type=(jax.ShapeDtypeStruct((ec, _D), jnp.float32),
                  jax.ShapeDtypeStruct((ec, _D), jnp.float32)),
        mesh=_sc_mesh(),
        scratch_types=[
            pltpu.VMEM((ipw, _C), jnp.int32),
            pltpu.VMEM((ipw, _C), jnp.int32),
            pltpu.VMEM((2, 2, _C, _D), jnp.float32),
            pltpu.SemaphoreType.DMA,
            pltpu.SemaphoreType.DMA,
        ],
    )
    def _sc_gather(n_hbm, s_hbm, r_hbm, sg_hbm, rg_hbm,
                   sidx, ridx, bufs, gsem, wsem):
        wid = lax.axis_index("s") * _NC + lax.axis_index("c")
        ebase = wid * ipw * _C
        pltpu.sync_copy(s_hbm.at[wid], sidx)
        pltpu.sync_copy(r_hbm.at[wid], ridx)

        def gath(j, b):
            pltpu.async_copy(n_hbm.at[sidx.at[j]], bufs.at[b, 0], gsem)
            pltpu.async_copy(n_hbm.at[ridx.at[j]], bufs.at[b, 1], gsem)

        def gath_wait(j, b):
            pltpu.make_async_copy(n_hbm.at[sidx.at[j]], bufs.at[b, 0], gsem).wait()
            pltpu.make_async_copy(n_hbm.at[ridx.at[j]], bufs.at[b, 1], gsem).wait()

        def wr(j, b):
            off = ebase + j * _C
            pltpu.async_copy(bufs.at[b, 0], sg_hbm.at[pl.ds(off, _C)], wsem)
            pltpu.async_copy(bufs.at[b, 1], rg_hbm.at[pl.ds(off, _C)], wsem)

        def wr_wait(j, b):
            off = ebase + j * _C
            pltpu.make_async_copy(bufs.at[b, 0], sg_hbm.at[pl.ds(off, _C)], wsem).wait()
            pltpu.make_async_copy(bufs.at[b, 1], rg_hbm.at[pl.ds(off, _C)], wsem).wait()

        gath(0, 0)

        def body(j2, carry):
            a = 2 * j2
            gath_wait(a, 0)

            @pl.when(j2 > 0)
            def _():
                wr_wait(a - 1, 1)

            wr(a, 0)
            gath(a + 1, 1)
            gath_wait(a + 1, 1)
            wr_wait(a, 0)
            wr(a + 1, 1)

            @pl.when(a + 2 < ipw)
            def _():
                gath(a + 2, 0)

            return carry

        lax.fori_loop(0, ipw // 2, body, 0)
        if ipw % 2:
            gath_wait(ipw - 1, 0)
            wr_wait(ipw - 2, 1)
            wr(ipw - 1, 0)
            wr_wait(ipw - 1, 0)
        else:
            wr_wait(ipw - 1, 1)

    return _sc_gather


@functools.cache
def _scatter_fn(ipw):
    ec = _csize(ipw)

    @functools.partial(
        pl.kernel,
        out_type=jax.ShapeDtypeStruct((_NC, _NPAD, _D), jnp.float32),
        mesh=_sc_mesh(),
        scratch_types=[
            pltpu.VMEM((ipw, _C), jnp.int32),
            pltpu.VMEM((2, _C, _D), jnp.float32),
            pltpu.VMEM_SHARED((_NPAD, _D), jnp.float32),
            pltpu.SemaphoreType.DMA,
        ],
    )
    def _sc_scatter(ue_hbm, r_hbm, out_hbm, ridx, ebuf, acc, rsem):
        cid = lax.axis_index("c")
        sid = lax.axis_index("s")
        wid = sid * _NC + cid

        def zrow(i, carry):
            for kk in range(_D // 16):
                ebuf[0, i, pl.ds(kk * 16, 16)] = jnp.zeros((16,), jnp.float32)
            return carry

        lax.fori_loop(0, _C, zrow, 0)
        for z in range(_RPT // _C):
            pltpu.sync_copy(ebuf.at[0], acc.at[pl.ds(sid * _RPT + z * _C, _C)])
        plsc.subcore_barrier()

        ebase = wid * ipw * _C
        pltpu.sync_copy(r_hbm.at[wid], ridx)

        def rd(j, b):
            pltpu.async_copy(ue_hbm.at[pl.ds(ebase + j * _C, _C)], ebuf.at[b], rsem)

        def rd_wait(j, b):
            pltpu.make_async_copy(ue_hbm.at[pl.ds(ebase + j * _C, _C)],
                                  ebuf.at[b], rsem).wait()

        rd(0, 0)

        def body(j2, carry):
            a = 2 * j2
            rd_wait(a, 0)
            rd(a + 1, 1)
            pltpu.sync_copy(ebuf.at[0], acc.at[ridx.at[a]], add=True)
            rd_wait(a + 1, 1)

            @pl.when(a + 2 < ipw)
            def _():
                rd(a + 2, 0)

            pltpu.sync_copy(ebuf.at[1], acc.at[ridx.at[a + 1]], add=True)
            return carry

        lax.fori_loop(0, ipw // 2, body, 0)
        if ipw % 2:
            rd_wait(ipw - 1, 0)
            pltpu.sync_copy(ebuf.at[0], acc.at[ridx.at[ipw - 1]], add=True)
        plsc.subcore_barrier()
        pltpu.sync_copy(acc.at[pl.ds(sid * _RPT, _RPT)],
                        out_hbm.at[cid, pl.ds(sid * _RPT, _RPT)])

    return _sc_scatter


def kernel(nodes, edges, senders, receivers, params):
    senders = senders.astype(jnp.int32)
    receivers = receivers.astype(jnp.int32)
    sidx = [senders[b:b + _csize(ipw)].reshape(_NW, ipw, _C) for b, ipw in _CHUNKS]
    ridx = [receivers[b:b + _csize(ipw)].reshape(_NW, ipw, _C) for b, ipw in _CHUNKS]

    n = _enc_mlp(nodes, params['enc_node'], _NBLK)

    nsteps = len(params['proc'])
    e_chunks = None
    for si, step in enumerate(params['proc']):
        want_e = si + 1 < nsteps
        aggs = []
        new_e_chunks = []
        for q, (b, ipw) in enumerate(_CHUNKS):
            sg, rg = _gather_fn(ipw)(n, sidx[q], ridx[q])
            if e_chunks is None:
                res = _edge_mlp(sg, rg, edges, b // _EBLK, step['edge'],
                                want_e, enc_p=params['enc_edge'])
            else:
                res = _edge_mlp(sg, rg, e_chunks[q], 0, step['edge'], want_e)
            if want_e:
                ue, en = res
                new_e_chunks.append(en)
            else:
                ue = res
            aggs.append(_scatter_fn(ipw)(ue, ridx[q]))
        e_chunks = new_e_chunks if want_e else e_chunks
        n = _node_mlp(n, aggs, step['node'])

    return _dec_mlp(n, params['dec'])


# speedup vs baseline: 3.1737x; 1.1029x over previous
import functools

import jax
import jax.numpy as jnp
from jax import lax
from jax.experimental import pallas as pl
from jax.experimental.pallas import tpu as pltpu
from jax.experimental.pallas import tpu_sc as plsc

_N = 10000
_E = 320000
_D = 128
_NC = 2
_NS = 16
_NW = _NC * _NS
_C = 80
_NPAD = 10240
_RPT = _NPAD // _NS

_NBLK = 1000
_EBLK = 1600

_CHUNKS = ((0, 60), (153600, 65))


def _csize(ipw):
    return _NW * ipw * _C


def _dot(x, w):
    return lax.dot_general(x, w, (((1,), (0,)), ((), ())),
                           preferred_element_type=jnp.float32)


def _ln(x, g, beta):
    mu = jnp.mean(x, axis=-1, keepdims=True)
    var = jnp.mean((x - mu) ** 2, axis=-1, keepdims=True)
    return (x - mu) * lax.rsqrt(var + 1e-5) * g + beta


def _hidden(h, w, b, g, beta):
    return jax.nn.relu(_ln(_dot(h, w) + b, g, beta))


def _full(shape):
    return pl.BlockSpec(shape, lambda i: tuple(0 for _ in shape))


def _rows(blk, d, off=0):
    return pl.BlockSpec((blk, d), lambda i: (off + i, 0))


def _prep_mlp(p):
    out = []
    for lyr in p:
        out.append(lyr['W'])
        out.append(lyr['b'][None, :])
        if 'g' in lyr:
            out.append(lyr['g'][None, :])
            out.append(lyr['beta'][None, :])
    return out


def _mlp_ln_body(x, w1, b1, g1, be1, w2, b2, g2, be2, w3, b3, pws, pwr,
                 o, ons, onr):
    h = _hidden(x[...], w1[...], b1[...], g1[...], be1[...])
    h = _hidden(h, w2[...], b2[...], g2[...], be2[...])
    out = _dot(h, w3[...]) + b3[...]
    o[...] = out
    ons[...] = _dot(out, pws[...])
    onr[...] = _dot(out, pwr[...])


def _enc_mlp(x, p, blk, pws, pwr):
    n, din = x.shape
    ws = _prep_mlp(p) + [pws, pwr]
    return pl.pallas_call(
        _mlp_ln_body,
        grid=(n // blk,),
        in_specs=[_rows(blk, din)] + [_full(w.shape) for w in ws],
        out_specs=[_rows(blk, _D)] * 3,
        out_shape=[jax.ShapeDtypeStruct((n, _D), jnp.float32)] * 3,
    )(x, *ws)


def _edge_body(want_e, n_enc, g, e, *refs):
    if n_enc:
        encw, refs = refs[:n_enc], refs[n_enc:]
        e0 = e[...]
        e0 = _hidden(e0, encw[0][...], encw[1][...], encw[2][...], encw[3][...])
        e0 = _hidden(e0, encw[4][...], encw[5][...], encw[6][...], encw[7][...])
        e0 = _dot(e0, encw[8][...]) + encw[9][...]
    else:
        e0 = e[...]
    (we, b1, g1, be1, w2, b2, g2, be2, w3, b3), refs = refs[:10], refs[10:]
    h = g[...] + _dot(e0, we[...]) + b1[...]
    h = jax.nn.relu(_ln(h, g1[...], be1[...]))
    h = _hidden(h, w2[...], b2[...], g2[...], be2[...])
    ue = _dot(h, w3[...]) + b3[...]
    refs[0][...] = ue
    if want_e:
        refs[1][...] = e0 + ue


def _edge_mlp(g, e, e_off, p, want_e, enc_p=None):
    ec = g.shape[0]
    ed = e.shape[1]
    encw = _prep_mlp(enc_p) if enc_p is not None else []
    w1 = p[0]['W']
    ws_ = encw + [w1[2 * _D:],
                  p[0]['b'][None, :], p[0]['g'][None, :], p[0]['beta'][None, :],
                  p[1]['W'], p[1]['b'][None, :], p[1]['g'][None, :], p[1]['beta'][None, :],
                  p[2]['W'], p[2]['b'][None, :]]
    out_shape = [jax.ShapeDtypeStruct((ec, _D), jnp.float32)]
    out_specs = [_rows(_EBLK, _D)]
    if want_e:
        out_shape.append(jax.ShapeDtypeStruct((ec, _D), jnp.float32))
        out_specs.append(_rows(_EBLK, _D))
    res = pl.pallas_call(
        functools.partial(_edge_body, want_e, len(encw)),
        grid=(ec // _EBLK,),
        in_specs=[_rows(_EBLK, _D), _rows(_EBLK, ed, e_off)]
                 + [_full(w.shape) for w in ws_],
        out_specs=out_specs,
        out_shape=out_shape,
    )(g, e, *ws_)
    return res if want_e else res[0]


def _node_body(proj, n, a0, a1, wn, wa, b1, g1, be1, w2, b2, g2, be2, w3, b3,
               *rest):
    agg = a0[0] + a0[1] + a1[0] + a1[1]
    h = _dot(n[...], wn[...]) + _dot(agg, wa[...]) + b1[...]
    h = jax.nn.relu(_ln(h, g1[...], be1[...]))
    h = _hidden(h, w2[...], b2[...], g2[...], be2[...])
    out = n[...] + _dot(h, w3[...]) + b3[...]
    if proj:
        pws, pwr, o, ons, onr = rest
        o[...] = out
        ons[...] = _dot(out, pws[...])
        onr[...] = _dot(out, pwr[...])
    else:
        rest[0][...] = out


def _node_mlp(n, aggs, p, proj=None):
    w1 = p[0]['W']
    ws_ = [w1[:_D], w1[_D:],
           p[0]['b'][None, :], p[0]['g'][None, :], p[0]['beta'][None, :],
           p[1]['W'], p[1]['b'][None, :], p[1]['g'][None, :], p[1]['beta'][None, :],
           p[2]['W'], p[2]['b'][None, :]]
    nout = 1
    if proj is not None:
        ws_ += [proj[0], proj[1]]
        nout = 3
    agg_spec = pl.BlockSpec((_NC, _NBLK, _D), lambda i: (0, i, 0))
    return pl.pallas_call(
        functools.partial(_node_body, proj is not None),
        grid=(_N // _NBLK,),
        in_specs=[_rows(_NBLK, _D)] + [agg_spec] * len(aggs)
                 + [_full(w.shape) for w in ws_],
        out_specs=[_rows(_NBLK, _D)] * nout,
        out_shape=[jax.ShapeDtypeStruct((_N, _D), jnp.float32)] * nout,
    )(n, *aggs, *ws_)


def _dec_body(x, w1, b1, w2, b2, w3, b3, o):
    h = jax.nn.relu(_dot(x[...], w1[...]) + b1[...])
    h = jax.nn.relu(_dot(h, w2[...]) + b2[...])
    o[...] = _dot(h, w3[...]) + b3[...]


def _dec_mlp(x, p):
    w3 = jnp.pad(p[2]['W'], ((0, 0), (0, _D - p[2]['W'].shape[1])))
    b3 = jnp.pad(p[2]['b'], (0, _D - p[2]['b'].shape[0]))[None, :]
    ws_ = [p[0]['W'], p[0]['b'][None, :], p[1]['W'], p[1]['b'][None, :], w3, b3]
    out = pl.pallas_call(
        _dec_body,
        grid=(_N // _NBLK,),
        in_specs=[_rows(_NBLK, _D)] + [_full(w.shape) for w in ws_],
        out_specs=_rows(_NBLK, _D),
        out_shape=jax.ShapeDtypeStruct((_N, _D), jnp.float32),
    )(x, *ws_)
    return out[:, :p[2]['W'].shape[1]]


def _sc_mesh():
    return plsc.VectorSubcoreMesh(core_axis_name="c", subcore_axis_name="s",
                                  num_cores=_NC, num_subcores=_NS)


@functools.cache
def _gather_fn(ipw):
    ec = _csize(ipw)

    @functools.partial(
        pl.kernel,
        out_type=jax.ShapeDtypeStruct((ec, _D), jnp.float32),
        mesh=_sc_mesh(),
        scratch_types=[
            pltpu.VMEM((ipw, _C), jnp.int32),
            pltpu.VMEM((ipw, _C), jnp.int32),
            pltpu.VMEM((2, 2, _C, _D), jnp.float32),
            pltpu.SemaphoreType.DMA,
            pltpu.SemaphoreType.DMA,
        ],
    )
    def _sc_gather(ns_hbm, nr_hbm, s_hbm, r_hbm, g_hbm,
                   sidx, ridx, bufs, gsem, wsem):
        wid = lax.axis_index("s") * _NC + lax.axis_index("c")
        ebase = wid * ipw * _C
        pltpu.sync_copy(s_hbm.at[wid], sidx)
        pltpu.sync_copy(r_hbm.at[wid], ridx)

        def gath(j, b):
            pltpu.async_copy(ns_hbm.at[sidx.at[j]], bufs.at[b, 0], gsem)
            pltpu.async_copy(nr_hbm.at[ridx.at[j]], bufs.at[b, 1], gsem)

        def gath_wait(j, b):
            pltpu.make_async_copy(ns_hbm.at[sidx.at[j]], bufs.at[b, 0], gsem).wait()
            pltpu.make_async_copy(nr_hbm.at[ridx.at[j]], bufs.at[b, 1], gsem).wait()

        def add(b):
            def arow(i, carry):
                for kk in range(_D // 16):
                    sl = pl.ds(kk * 16, 16)
                    bufs[b, 0, i, sl] = bufs[b, 0, i, sl] + bufs[b, 1, i, sl]
                return carry

            lax.fori_loop(0, _C, arow, 0)

        def wr(j, b):
            pltpu.async_copy(bufs.at[b, 0],
                             g_hbm.at[pl.ds(ebase + j * _C, _C)], wsem)

        def wr_wait(j, b):
            pltpu.make_async_copy(bufs.at[b, 0],
                                  g_hbm.at[pl.ds(ebase + j * _C, _C)], wsem).wait()

        gath(0, 0)

        def body(j2, carry):
            a = 2 * j2
            gath_wait(a, 0)

            @pl.when(j2 > 0)
            def _():
                wr_wait(a - 1, 1)

            gath(a + 1, 1)
            add(0)
            wr(a, 0)
            gath_wait(a + 1, 1)
            wr_wait(a, 0)
            add(1)
            wr(a + 1, 1)

            @pl.when(a + 2 < ipw)
            def _():
                gath(a + 2, 0)

            return carry

        lax.fori_loop(0, ipw // 2, body, 0)
        if ipw % 2:
            gath_wait(ipw - 1, 0)
            wr_wait(ipw - 2, 1)
            add(0)
            wr(ipw - 1, 0)
            wr_wait(ipw - 1, 0)
        else:
            wr_wait(ipw - 1, 1)

    return _sc_gather


@functools.cache
def _scatter_fn(ipw):
    ec = _csize(ipw)

    @functools.partial(
        pl.kernel,
        out_type=jax.ShapeDtypeStruct((_NC, _NPAD, _D), jnp.float32),
        mesh=_sc_mesh(),
        scratch_types=[
            pltpu.VMEM((ipw, _C), jnp.int32),
            pltpu.VMEM((2, _C, _D), jnp.float32),
            pltpu.VMEM_SHARED((_NPAD, _D), jnp.float32),
            pltpu.SemaphoreType.DMA,
        ],
    )
    def _sc_scatter(ue_hbm, r_hbm, out_hbm, ridx, ebuf, acc, rsem):
        cid = lax.axis_index("c")
        sid = lax.axis_index("s")
        wid = sid * _NC + cid

        def zrow(i, carry):
            for kk in range(_D // 16):
                ebuf[0, i, pl.ds(kk * 16, 16)] = jnp.zeros((16,), jnp.float32)
            return carry

        lax.fori_loop(0, _C, zrow, 0)
        for z in range(_RPT // _C):
            pltpu.sync_copy(ebuf.at[0], acc.at[pl.ds(sid * _RPT + z * _C, _C)])
        plsc.subcore_barrier()

        ebase = wid * ipw * _C
        pltpu.sync_copy(r_hbm.at[wid], ridx)

        def rd(j, b):
            pltpu.async_copy(ue_hbm.at[pl.ds(ebase + j * _C, _C)], ebuf.at[b], rsem)

        def rd_wait(j, b):
            pltpu.make_async_copy(ue_hbm.at[pl.ds(ebase + j * _C, _C)],
                                  ebuf.at[b], rsem).wait()

        rd(0, 0)

        def body(j2, carry):
            a = 2 * j2
            rd_wait(a, 0)
            rd(a + 1, 1)
            pltpu.sync_copy(ebuf.at[0], acc.at[ridx.at[a]], add=True)
            rd_wait(a + 1, 1)

            @pl.when(a + 2 < ipw)
            def _():
                rd(a + 2, 0)

            pltpu.sync_copy(ebuf.at[1], acc.at[ridx.at[a + 1]], add=True)
            return carry

        lax.fori_loop(0, ipw // 2, body, 0)
        if ipw % 2:
            rd_wait(ipw - 1, 0)
            pltpu.sync_copy(ebuf.at[0], acc.at[ridx.at[ipw - 1]], add=True)
        plsc.subcore_barrier()
        pltpu.sync_copy(acc.at[pl.ds(sid * _RPT, _RPT)],
                        out_hbm.at[cid, pl.ds(sid * _RPT, _RPT)])

    return _sc_scatter


def kernel(nodes, edges, senders, receivers, params):
    senders = senders.astype(jnp.int32)
    receivers = receivers.astype(jnp.int32)
    sidx = [senders[b:b + _csize(ipw)].reshape(_NW, ipw, _C) for b, ipw in _CHUNKS]
    ridx = [receivers[b:b + _csize(ipw)].reshape(_NW, ipw, _C) for b, ipw in _CHUNKS]

    def _proj_w(step):
        w1 = step['edge'][0]['W']
        return w1[:_D], w1[_D:2 * _D]

    n, ns, nr = _enc_mlp(nodes, params['enc_node'], _NBLK,
                         *_proj_w(params['proc'][0]))

    nsteps = len(params['proc'])
    e_chunks = None
    for si, step in enumerate(params['proc']):
        want_e = si + 1 < nsteps
        aggs = []
        new_e_chunks = []
        for q, (b, ipw) in enumerate(_CHUNKS):
            g = _gather_fn(ipw)(ns, nr, sidx[q], ridx[q])
            if e_chunks is None:
                res = _edge_mlp(g, edges, b // _EBLK, step['edge'],
                                want_e, enc_p=params['enc_edge'])
            else:
                res = _edge_mlp(g, e_chunks[q], 0, step['edge'], want_e)
            if want_e:
                ue, en = res
                new_e_chunks.append(en)
            else:
                ue = res
            aggs.append(_scatter_fn(ipw)(ue, ridx[q]))
        e_chunks = new_e_chunks if want_e else e_chunks
        if want_e:
            n, ns, nr = _node_mlp(n, aggs, step['node'],
                                  proj=_proj_w(params['proc'][si + 1]))
        else:
            n = _node_mlp(n, aggs, step['node'])[0]

    return _dec_mlp(n, params['dec'])


# speedup vs baseline: 3.3881x; 1.0676x over previous
import functools

import jax
import jax.numpy as jnp
from jax import lax
from jax.experimental import pallas as pl
from jax.experimental.pallas import tpu as pltpu
from jax.experimental.pallas import tpu_sc as plsc

_N = 10000
_E = 320000
_D = 128
_NC = 2
_NS = 16
_NW = _NC * _NS
_C = 80
_NPAD = 10240
_RPT = _NPAD // _NS

_NBLK = 1000
_EBLK = 1600

_CHUNKS = ((0, 40), (102400, 40), (204800, 45))


def _csize(ipw):
    return _NW * ipw * _C


def _dot(x, w):
    return lax.dot_general(x, w, (((1,), (0,)), ((), ())),
                           preferred_element_type=jnp.float32)


def _ln(x, g, beta):
    mu = jnp.mean(x, axis=-1, keepdims=True)
    var = jnp.mean((x - mu) ** 2, axis=-1, keepdims=True)
    return (x - mu) * lax.rsqrt(var + 1e-5) * g + beta


def _hidden(h, w, b, g, beta):
    return jax.nn.relu(_ln(_dot(h, w) + b, g, beta))


def _full(shape):
    return pl.BlockSpec(shape, lambda i: tuple(0 for _ in shape))


def _rows(blk, d, off=0):
    return pl.BlockSpec((blk, d), lambda i: (off + i, 0))


def _prep_mlp(p):
    out = []
    for lyr in p:
        out.append(lyr['W'])
        out.append(lyr['b'][None, :])
        if 'g' in lyr:
            out.append(lyr['g'][None, :])
            out.append(lyr['beta'][None, :])
    return out


def _mlp_ln_body(x, w1, b1, g1, be1, w2, b2, g2, be2, w3, b3, pws, pwr,
                 o, ons, onr):
    h = _hidden(x[...], w1[...], b1[...], g1[...], be1[...])
    h = _hidden(h, w2[...], b2[...], g2[...], be2[...])
    out = _dot(h, w3[...]) + b3[...]
    o[...] = out
    ons[...] = _dot(out, pws[...])
    onr[...] = _dot(out, pwr[...])


def _enc_mlp(x, p, blk, pws, pwr):
    n, din = x.shape
    ws = _prep_mlp(p) + [pws, pwr]
    return pl.pallas_call(
        _mlp_ln_body,
        grid=(n // blk,),
        in_specs=[_rows(blk, din)] + [_full(w.shape) for w in ws],
        out_specs=[_rows(blk, _D)] * 3,
        out_shape=[jax.ShapeDtypeStruct((n, _D), jnp.float32)] * 3,
    )(x, *ws)


def _edge_body(want_e, n_enc, g, e, *refs):
    if n_enc:
        encw, refs = refs[:n_enc], refs[n_enc:]
        e0 = e[...]
        e0 = _hidden(e0, encw[0][...], encw[1][...], encw[2][...], encw[3][...])
        e0 = _hidden(e0, encw[4][...], encw[5][...], encw[6][...], encw[7][...])
        e0 = _dot(e0, encw[8][...]) + encw[9][...]
    else:
        e0 = e[...]
    (we, b1, g1, be1, w2, b2, g2, be2, w3, b3), refs = refs[:10], refs[10:]
    h = g[...] + _dot(e0, we[...]) + b1[...]
    h = jax.nn.relu(_ln(h, g1[...], be1[...]))
    h = _hidden(h, w2[...], b2[...], g2[...], be2[...])
    ue = _dot(h, w3[...]) + b3[...]
    refs[0][...] = ue
    if want_e:
        refs[1][...] = e0 + ue


def _edge_mlp(g, e, e_off, p, want_e, enc_p=None):
    ec = g.shape[0]
    ed = e.shape[1]
    encw = _prep_mlp(enc_p) if enc_p is not None else []
    w1 = p[0]['W']
    ws_ = encw + [w1[2 * _D:],
                  p[0]['b'][None, :], p[0]['g'][None, :], p[0]['beta'][None, :],
                  p[1]['W'], p[1]['b'][None, :], p[1]['g'][None, :], p[1]['beta'][None, :],
                  p[2]['W'], p[2]['b'][None, :]]
    out_shape = [jax.ShapeDtypeStruct((ec, _D), jnp.float32)]
    out_specs = [_rows(_EBLK, _D)]
    if want_e:
        out_shape.append(jax.ShapeDtypeStruct((ec, _D), jnp.float32))
        out_specs.append(_rows(_EBLK, _D))
    res = pl.pallas_call(
        functools.partial(_edge_body, want_e, len(encw)),
        grid=(ec // _EBLK,),
        in_specs=[_rows(_EBLK, _D), _rows(_EBLK, ed, e_off)]
                 + [_full(w.shape) for w in ws_],
        out_specs=out_specs,
        out_shape=out_shape,
    )(g, e, *ws_)
    return res if want_e else res[0]


def _node_body(proj, nagg, n, *rest):
    aggs, rest = rest[:nagg], rest[nagg:]
    (wn, wa, b1, g1, be1, w2, b2, g2, be2, w3, b3), rest = rest[:11], rest[11:]
    agg = sum(a[0] + a[1] for a in aggs)
    h = _dot(n[...], wn[...]) + _dot(agg, wa[...]) + b1[...]
    h = jax.nn.relu(_ln(h, g1[...], be1[...]))
    h = _hidden(h, w2[...], b2[...], g2[...], be2[...])
    out = n[...] + _dot(h, w3[...]) + b3[...]
    if proj:
        pws, pwr, o, ons, onr = rest
        o[...] = out
        ons[...] = _dot(out, pws[...])
        onr[...] = _dot(out, pwr[...])
    else:
        rest[0][...] = out


def _node_mlp(n, aggs, p, proj=None):
    w1 = p[0]['W']
    ws_ = [w1[:_D], w1[_D:],
           p[0]['b'][None, :], p[0]['g'][None, :], p[0]['beta'][None, :],
           p[1]['W'], p[1]['b'][None, :], p[1]['g'][None, :], p[1]['beta'][None, :],
           p[2]['W'], p[2]['b'][None, :]]
    nout = 1
    if proj is not None:
        ws_ += [proj[0], proj[1]]
        nout = 3
    agg_spec = pl.BlockSpec((_NC, _NBLK, _D), lambda i: (0, i, 0))
    return pl.pallas_call(
        functools.partial(_node_body, proj is not None, len(aggs)),
        grid=(_N // _NBLK,),
        in_specs=[_rows(_NBLK, _D)] + [agg_spec] * len(aggs)
                 + [_full(w.shape) for w in ws_],
        out_specs=[_rows(_NBLK, _D)] * nout,
        out_shape=[jax.ShapeDtypeStruct((_N, _D), jnp.float32)] * nout,
    )(n, *aggs, *ws_)


def _dec_body(x, w1, b1, w2, b2, w3, b3, o):
    h = jax.nn.relu(_dot(x[...], w1[...]) + b1[...])
    h = jax.nn.relu(_dot(h, w2[...]) + b2[...])
    o[...] = _dot(h, w3[...]) + b3[...]


def _dec_mlp(x, p):
    w3 = jnp.pad(p[2]['W'], ((0, 0), (0, _D - p[2]['W'].shape[1])))
    b3 = jnp.pad(p[2]['b'], (0, _D - p[2]['b'].shape[0]))[None, :]
    ws_ = [p[0]['W'], p[0]['b'][None, :], p[1]['W'], p[1]['b'][None, :], w3, b3]
    out = pl.pallas_call(
        _dec_body,
        grid=(_N // _NBLK,),
        in_specs=[_rows(_NBLK, _D)] + [_full(w.shape) for w in ws_],
        out_specs=_rows(_NBLK, _D),
        out_shape=jax.ShapeDtypeStruct((_N, _D), jnp.float32),
    )(x, *ws_)
    return out[:, :p[2]['W'].shape[1]]


def _sc_mesh():
    return plsc.VectorSubcoreMesh(core_axis_name="c", subcore_axis_name="s",
                                  num_cores=_NC, num_subcores=_NS)


@functools.cache
def _gather_fn(ipw):
    ec = _csize(ipw)

    @functools.partial(
        pl.kernel,
        out_type=jax.ShapeDtypeStruct((ec, _D), jnp.float32),
        mesh=_sc_mesh(),
        scratch_types=[
            pltpu.VMEM((ipw, _C), jnp.int32),
            pltpu.VMEM((ipw, _C), jnp.int32),
            pltpu.VMEM((2, 2, _C, _D), jnp.float32),
            pltpu.SemaphoreType.DMA,
            pltpu.SemaphoreType.DMA,
        ],
    )
    def _sc_gather(ns_hbm, nr_hbm, s_hbm, r_hbm, g_hbm,
                   sidx, ridx, bufs, gsem, wsem):
        wid = lax.axis_index("s") * _NC + lax.axis_index("c")
        ebase = wid * ipw * _C
        pltpu.sync_copy(s_hbm.at[wid], sidx)
        pltpu.sync_copy(r_hbm.at[wid], ridx)

        def gath(j, b):
            pltpu.async_copy(ns_hbm.at[sidx.at[j]], bufs.at[b, 0], gsem)
            pltpu.async_copy(nr_hbm.at[ridx.at[j]], bufs.at[b, 1], gsem)

        def gath_wait(j, b):
            pltpu.make_async_copy(ns_hbm.at[sidx.at[j]], bufs.at[b, 0], gsem).wait()
            pltpu.make_async_copy(nr_hbm.at[ridx.at[j]], bufs.at[b, 1], gsem).wait()

        def add(b):
            def arow(i, carry):
                for kk in range(_D // 16):
                    sl = pl.ds(kk * 16, 16)
                    bufs[b, 0, i, sl] = bufs[b, 0, i, sl] + bufs[b, 1, i, sl]
                return carry

            lax.fori_loop(0, _C, arow, 0)

        def wr(j, b):
            pltpu.async_copy(bufs.at[b, 0],
                             g_hbm.at[pl.ds(ebase + j * _C, _C)], wsem)

        def wr_wait(j, b):
            pltpu.make_async_copy(bufs.at[b, 0],
                                  g_hbm.at[pl.ds(ebase + j * _C, _C)], wsem).wait()

        gath(0, 0)

        def body(j2, carry):
            a = 2 * j2
            gath_wait(a, 0)

            @pl.when(j2 > 0)
            def _():
                wr_wait(a - 1, 1)

            gath(a + 1, 1)
            add(0)
            wr(a, 0)
            gath_wait(a + 1, 1)
            wr_wait(a, 0)
            add(1)
            wr(a + 1, 1)

            @pl.when(a + 2 < ipw)
            def _():
                gath(a + 2, 0)

            return carry

        lax.fori_loop(0, ipw // 2, body, 0)
        if ipw % 2:
            gath_wait(ipw - 1, 0)
            wr_wait(ipw - 2, 1)
            add(0)
            wr(ipw - 1, 0)
            wr_wait(ipw - 1, 0)
        else:
            wr_wait(ipw - 1, 1)

    return _sc_gather


@functools.cache
def _scatter_fn(ipw):
    ec = _csize(ipw)

    @functools.partial(
        pl.kernel,
        out_type=jax.ShapeDtypeStruct((_NC, _NPAD, _D), jnp.float32),
        mesh=_sc_mesh(),
        scratch_types=[
            pltpu.VMEM((ipw, _C), jnp.int32),
            pltpu.VMEM((2, _C, _D), jnp.float32),
            pltpu.VMEM_SHARED((_NPAD, _D), jnp.float32),
            pltpu.SemaphoreType.DMA,
        ],
    )
    def _sc_scatter(ue_hbm, r_hbm, out_hbm, ridx, ebuf, acc, rsem):
        cid = lax.axis_index("c")
        sid = lax.axis_index("s")
        wid = sid * _NC + cid

        def zrow(i, carry):
            for kk in range(_D // 16):
                ebuf[0, i, pl.ds(kk * 16, 16)] = jnp.zeros((16,), jnp.float32)
            return carry

        lax.fori_loop(0, _C, zrow, 0)
        for z in range(_RPT // _C):
            pltpu.sync_copy(ebuf.at[0], acc.at[pl.ds(sid * _RPT + z * _C, _C)])
        plsc.subcore_barrier()

        ebase = wid * ipw * _C
        pltpu.sync_copy(r_hbm.at[wid], ridx)

        def rd(j, b):
            pltpu.async_copy(ue_hbm.at[pl.ds(ebase + j * _C, _C)], ebuf.at[b], rsem)

        def rd_wait(j, b):
            pltpu.make_async_copy(ue_hbm.at[pl.ds(ebase + j * _C, _C)],
                                  ebuf.at[b], rsem).wait()

        rd(0, 0)

        def body(j2, carry):
            a = 2 * j2
            rd_wait(a, 0)
            rd(a + 1, 1)
            pltpu.sync_copy(ebuf.at[0], acc.at[ridx.at[a]], add=True)
            rd_wait(a + 1, 1)

            @pl.when(a + 2 < ipw)
            def _():
                rd(a + 2, 0)

            pltpu.sync_copy(ebuf.at[1], acc.at[ridx.at[a + 1]], add=True)
            return carry

        lax.fori_loop(0, ipw // 2, body, 0)
        if ipw % 2:
            rd_wait(ipw - 1, 0)
            pltpu.sync_copy(ebuf.at[0], acc.at[ridx.at[ipw - 1]], add=True)
        plsc.subcore_barrier()
        pltpu.sync_copy(acc.at[pl.ds(sid * _RPT, _RPT)],
                        out_hbm.at[cid, pl.ds(sid * _RPT, _RPT)])

    return _sc_scatter


def kernel(nodes, edges, senders, receivers, params):
    senders = senders.astype(jnp.int32)
    receivers = receivers.astype(jnp.int32)
    sidx = [senders[b:b + _csize(ipw)].reshape(_NW, ipw, _C) for b, ipw in _CHUNKS]
    ridx = [receivers[b:b + _csize(ipw)].reshape(_NW, ipw, _C) for b, ipw in _CHUNKS]

    def _proj_w(step):
        w1 = step['edge'][0]['W']
        return w1[:_D], w1[_D:2 * _D]

    n, ns, nr = _enc_mlp(nodes, params['enc_node'], _NBLK,
                         *_proj_w(params['proc'][0]))

    nsteps = len(params['proc'])
    e_chunks = None
    for si, step in enumerate(params['proc']):
        want_e = si + 1 < nsteps
        aggs = []
        new_e_chunks = []
        for q, (b, ipw) in enumerate(_CHUNKS):
            g = _gather_fn(ipw)(ns, nr, sidx[q], ridx[q])
            if e_chunks is None:
                res = _edge_mlp(g, edges, b // _EBLK, step['edge'],
                                want_e, enc_p=params['enc_edge'])
            else:
                res = _edge_mlp(g, e_chunks[q], 0, step['edge'], want_e)
            if want_e:
                ue, en = res
                new_e_chunks.append(en)
            else:
                ue = res
            aggs.append(_scatter_fn(ipw)(ue, ridx[q]))
        e_chunks = new_e_chunks if want_e else e_chunks
        if want_e:
            n, ns, nr = _node_mlp(n, aggs, step['node'],
                                  proj=_proj_w(params['proc'][si + 1]))
        else:
            n = _node_mlp(n, aggs, step['node'])[0]

    return _dec_mlp(n, params['dec'])
